# Initial kernel scaffold; baseline (speedup 1.0000x reference)
import jax
import jax.numpy as jnp
from jax.experimental import pallas as pl


def kernel(x, conv1, conv2, seg_logits, conv_parameter, sel_idx):
    raise NotImplementedError("write your pallas kernel here")


# speedup vs baseline: 6.0938x; 6.0938x over previous
import functools
import jax
import jax.numpy as jnp
from jax import lax
from jax.experimental import pallas as pl
from jax.experimental.pallas import tpu as pltpu
from jax.experimental.pallas import tpu_sc as plsc

B = 4
N = 32
N2 = N * N
K = 131
KP = 144
NCH = KP // 16
NC = 21
NCP = 32
RAD = 3
NOFF = 49
NSEL = 307
NSELP = 320
HALO = 128
ROWS = 128 + 2 * HALO
NBLK = 8
FT_W = ROWS * KP
SLAB_W = ROWS * NCP
BLK_W = 128 * NCP


def _sc_affinity(featT, x2T, selp, param):
    mesh = plsc.VectorSubcoreMesh(core_axis_name="c", subcore_axis_name="s")

    @functools.partial(
        pl.kernel,
        out_type=jax.ShapeDtypeStruct((B, N2 * NCP), jnp.float32),
        mesh=mesh,
        scratch_types=[
            pltpu.VMEM((FT_W,), jnp.float32),
            pltpu.VMEM((128 * NCP,), jnp.float32),
            pltpu.VMEM((NSELP,), jnp.int32),
            pltpu.SMEM((512,), jnp.float32),
            pltpu.VMEM((KP,), jnp.float32),
            pltpu.VMEM((NOFF * KP,), jnp.float32),
            pltpu.VMEM((SLAB_W,), jnp.float32),
            pltpu.VMEM((2 * BLK_W,), jnp.float32),
            pltpu.VMEM((BLK_W,), jnp.float32),
            pltpu.VMEM_SHARED((16 * SLAB_W,), jnp.float32),
        ],
    )
    def k(featT_hbm, x2T_hbm, selp_hbm, param_hbm, out_hbm,
          ft, x2v, selv, msmem, pv, ev, yloc, ybuf, yout, ysh):
        sid = lax.axis_index("s")
        b_loc = jnp.remainder(sid, 2)
        b = lax.axis_index("c") * 2 + b_loc
        rb = sid // 2
        start = rb * 128

        zeros16 = jnp.zeros((16,), jnp.float32)

        pltpu.sync_copy(featT_hbm.at[b].at[pl.ds(start * KP, FT_W)], ft)
        pltpu.sync_copy(x2T_hbm.at[b].at[pl.ds(start * NCP, 128 * NCP)], x2v)
        pltpu.sync_copy(selp_hbm.at[b], selv)
        pltpu.sync_copy(param_hbm, pv)

        def zrow(i, _):
            yloc[pl.ds(i * 16, 16)] = zeros16
            return 0
        lax.fori_loop(0, SLAB_W // 16, zrow, 0)

        def mzero(i, _):
            msmem[i] = 0.0
            return 0
        lax.fori_loop(0, ROWS, mzero, 0)
        for j in range(NSELP // 16):
            idxv = selv[pl.ds(j * 16, 16)]
            for t in range(16):
                l = idxv[t] - start + HALO
                ok = (l >= 0) & (l < ROWS)
                msmem[jnp.where(ok, l, ROWS + 32)] = 1.0

        def pix(q_local, _):
            q = start + q_local
            mq = msmem[q_local + HALO]

            @pl.when(mq > 0.0)
            def _():
                lrow = q_local + HALO
                fq = [ft[pl.ds(lrow * KP + j * 16, 16)] for j in range(NCH)]
                r = q // N
                c = jnp.remainder(q, N)

                def offA(o, rs):
                    dr = o // 7 - RAD
                    dc = jnp.remainder(o, 7) - RAD
                    q2 = q + dr * N + dc
                    valid = ((c + dc >= 0) & (c + dc < N)
                             & (r + dr >= 0) & (r + dr < N))
                    l2 = q2 - start + HALO
                    msk = jnp.where(valid, msmem[l2], 0.0)
                    out = []
                    for j in range(NCH):
                        fn = ft[pl.ds(l2 * KP + j * 16, 16)]
                        e = jnp.exp(-jnp.abs(fq[j] - fn)) * msk
                        ev[pl.ds(o * KP + j * 16, 16)] = e
                        out.append(rs[j] + e)
                    return tuple(out)

                rs = lax.fori_loop(
                    0, NOFF, offA,
                    tuple(jnp.zeros((16,), jnp.float32) for _ in range(NCH)))

                inv = [pv[pl.ds(j * 16, 16)]
                       / jnp.where(rs[j] > 0.0, rs[j], 1.0)
                       for j in range(NCH)]
                x2a = x2v[pl.ds(q_local * NCP, 16)]
                x2b = x2v[pl.ds(q_local * NCP + 16, 16)]

                def offB(o, _c):
                    acc = ev[pl.ds(o * KP, 16)] * inv[0]
                    for j in range(1, NCH):
                        acc = acc + ev[pl.ds(o * KP + j * 16, 16)] * inv[j]
                    lanes = [acc[t] for t in range(16)]
                    while len(lanes) > 1:
                        lanes = [a2 + b2 for a2, b2 in
                                 zip(lanes[::2], lanes[1::2])]
                    w = lanes[0]
                    dr = o // 7 - RAD
                    dc = jnp.remainder(o, 7) - RAD
                    l2 = q_local + HALO + dr * N + dc
                    ya = yloc[pl.ds(l2 * NCP, 16)]
                    yb = yloc[pl.ds(l2 * NCP + 16, 16)]
                    yloc[pl.ds(l2 * NCP, 16)] = ya + w * x2a
                    yloc[pl.ds(l2 * NCP + 16, 16)] = yb + w * x2b
                    return _c

                lax.fori_loop(0, NOFF, offB, 0)
            return 0

        lax.fori_loop(0, 128, pix, 0)

        pltpu.sync_copy(yloc, ysh.at[pl.ds(sid * SLAB_W, SLAB_W)])
        plsc.subcore_barrier()

        lo = jnp.maximum(sid - 2, 0)
        hi = jnp.minimum(sid + 2, 15)
        pltpu.sync_copy(ysh.at[pl.ds(lo * SLAB_W + 2 * BLK_W, BLK_W)],
                        ybuf.at[pl.ds(0, BLK_W)])
        pltpu.sync_copy(ysh.at[pl.ds(hi * SLAB_W, BLK_W)],
                        ybuf.at[pl.ds(BLK_W, BLK_W)])
        has_lo = rb > 0
        has_hi = rb < NBLK - 1

        def addrow(i, _):
            own = yloc[pl.ds(BLK_W + i * 16, 16)]
            va = jnp.where(has_lo, ybuf[pl.ds(i * 16, 16)], zeros16)
            vb = jnp.where(has_hi, ybuf[pl.ds(BLK_W + i * 16, 16)], zeros16)
            yout[pl.ds(i * 16, 16)] = own + va + vb
            return 0
        lax.fori_loop(0, BLK_W // 16, addrow, 0)

        pltpu.sync_copy(yout, out_hbm.at[b].at[pl.ds(start * NCP, BLK_W)])

    return k(featT, x2T, selp, param)


def kernel(x, conv1, conv2, seg_logits, conv_parameter, sel_idx):
    feat = jnp.concatenate([x, conv1, conv2], axis=1).reshape(B, K, N2)
    featT = jnp.transpose(feat, (0, 2, 1))
    featT = jnp.pad(featT, ((0, 0), (HALO, HALO), (0, KP - K)))
    featT = featT.reshape(B, (N2 + 2 * HALO) * KP)
    x2T = jnp.transpose(seg_logits.reshape(B, NC, N2), (0, 2, 1))
    x2T = jnp.pad(x2T, ((0, 0), (0, 0), (0, NCP - NC)))
    x2T = x2T.reshape(B, N2 * NCP)
    selp = jnp.concatenate(
        [sel_idx, jnp.tile(sel_idx[:, :1], (1, NSELP - NSEL))], axis=1)
    param = jnp.pad(conv_parameter, (0, KP - K))
    out = _sc_affinity(featT, x2T, selp.astype(jnp.int32), param)
    return out.reshape(B, N2, NCP)[:, :, :NC]


# speedup vs baseline: 10.0197x; 1.6442x over previous
import functools
import jax
import jax.numpy as jnp
from jax import lax
from jax.experimental import pallas as pl
from jax.experimental.pallas import tpu as pltpu
from jax.experimental.pallas import tpu_sc as plsc

B = 4
N = 32
N2 = N * N
K = 131
KP = 144
NCH = KP // 16
NC = 21
NCP = 32
RAD = 3
NOFF = 49
NSEL = 307
NSELP = 320
HALO = 128
ROWS = 128 + 2 * HALO
NBLK = 8
FT_W = ROWS * KP
SLAB_W = ROWS * NCP
BLK_W = 128 * NCP


def _sc_affinity(featT, x2T, selp, param):
    mesh = plsc.VectorSubcoreMesh(core_axis_name="c", subcore_axis_name="s")

    @functools.partial(
        pl.kernel,
        out_type=jax.ShapeDtypeStruct((B, N2 * NCP), jnp.float32),
        mesh=mesh,
        scratch_types=[
            pltpu.VMEM((FT_W,), jnp.float32),
            pltpu.VMEM((128 * NCP,), jnp.float32),
            pltpu.VMEM((NSELP,), jnp.int32),
            pltpu.SMEM((512,), jnp.float32),
            pltpu.VMEM((KP,), jnp.float32),
            pltpu.VMEM((NOFF * KP,), jnp.float32),
            pltpu.VMEM((KP,), jnp.float32),
            pltpu.VMEM((SLAB_W,), jnp.float32),
            pltpu.VMEM((2 * BLK_W,), jnp.float32),
            pltpu.VMEM((BLK_W,), jnp.float32),
            pltpu.VMEM_SHARED((16 * SLAB_W,), jnp.float32),
        ],
    )
    def k(featT_hbm, x2T_hbm, selp_hbm, param_hbm, out_hbm,
          ft, x2v, selv, msmem, pv, ev, rsv, yloc, ybuf, yout, ysh):
        sid = lax.axis_index("s")
        b_loc = jnp.remainder(sid, 2)
        b = lax.axis_index("c") * 2 + b_loc
        rb = sid // 2
        start = rb * 128

        zeros16 = jnp.zeros((16,), jnp.float32)

        pltpu.sync_copy(featT_hbm.at[b].at[pl.ds(start * KP, FT_W)], ft)
        pltpu.sync_copy(x2T_hbm.at[b].at[pl.ds(start * NCP, 128 * NCP)], x2v)
        pltpu.sync_copy(selp_hbm.at[b], selv)
        pltpu.sync_copy(param_hbm, pv)

        def zrow(i, _):
            yloc[pl.ds(i * 16, 16)] = zeros16
            return 0
        lax.fori_loop(0, SLAB_W // 16, zrow, 0)

        def mzero(i, _):
            msmem[i] = 0.0
            return 0
        lax.fori_loop(0, ROWS, mzero, 0)
        for j in range(NSELP // 16):
            idxv = selv[pl.ds(j * 16, 16)]
            for t in range(16):
                l = idxv[t] - start + HALO
                ok = (l >= 0) & (l < ROWS)
                msmem[jnp.where(ok, l, ROWS + 32)] = 1.0

        def pix(q_local, _):
            q = start + q_local
            mq = msmem[q_local + HALO]

            @pl.when(mq > 0.0)
            def _():
                lrow = q_local + HALO
                fq = [ft[pl.ds(lrow * KP + j * 16, 16)] for j in range(NCH)]
                r = q // N
                c = jnp.remainder(q, N)

                for j in range(NCH):
                    rsv[pl.ds(j * 16, 16)] = zeros16

                def offA(o, _c):
                    dr = o // 7 - RAD
                    dc = jnp.remainder(o, 7) - RAD
                    q2 = q + dr * N + dc
                    valid = ((c + dc >= 0) & (c + dc < N)
                             & (r + dr >= 0) & (r + dr < N))
                    l2 = q2 - start + HALO
                    msk = jnp.where(valid, msmem[l2], 0.0)
                    msmem[448 + o] = msk

                    @pl.when(msk > 0.0)
                    def _():
                        for j in range(NCH):
                            fn = ft[pl.ds(l2 * KP + j * 16, 16)]
                            e = jnp.exp(-jnp.abs(fq[j] - fn))
                            ev[pl.ds(o * KP + j * 16, 16)] = e
                            rsv[pl.ds(j * 16, 16)] = rsv[pl.ds(j * 16, 16)] + e
                    return _c

                lax.fori_loop(0, NOFF, offA, 0)

                rs = [rsv[pl.ds(j * 16, 16)] for j in range(NCH)]
                inv = [pv[pl.ds(j * 16, 16)]
                       / jnp.where(rs[j] > 0.0, rs[j], 1.0)
                       for j in range(NCH)]
                x2a = x2v[pl.ds(q_local * NCP, 16)]
                x2b = x2v[pl.ds(q_local * NCP + 16, 16)]

                def offB(o, _c):
                    @pl.when(msmem[448 + o] > 0.0)
                    def _():
                        acc = ev[pl.ds(o * KP, 16)] * inv[0]
                        for j in range(1, NCH):
                            acc = acc + ev[pl.ds(o * KP + j * 16, 16)] * inv[j]
                        lanes = [acc[t] for t in range(16)]
                        while len(lanes) > 1:
                            lanes = [a2 + b2 for a2, b2 in
                                     zip(lanes[::2], lanes[1::2])]
                        w = lanes[0]
                        dr = o // 7 - RAD
                        dc = jnp.remainder(o, 7) - RAD
                        l2 = q_local + HALO + dr * N + dc
                        ya = yloc[pl.ds(l2 * NCP, 16)]
                        yb = yloc[pl.ds(l2 * NCP + 16, 16)]
                        yloc[pl.ds(l2 * NCP, 16)] = ya + w * x2a
                        yloc[pl.ds(l2 * NCP + 16, 16)] = yb + w * x2b
                    return _c

                lax.fori_loop(0, NOFF, offB, 0)
            return 0

        lax.fori_loop(0, 128, pix, 0)

        pltpu.sync_copy(yloc, ysh.at[pl.ds(sid * SLAB_W, SLAB_W)])
        plsc.subcore_barrier()

        lo = jnp.maximum(sid - 2, 0)
        hi = jnp.minimum(sid + 2, 15)
        pltpu.sync_copy(ysh.at[pl.ds(lo * SLAB_W + 2 * BLK_W, BLK_W)],
                        ybuf.at[pl.ds(0, BLK_W)])
        pltpu.sync_copy(ysh.at[pl.ds(hi * SLAB_W, BLK_W)],
                        ybuf.at[pl.ds(BLK_W, BLK_W)])
        has_lo = rb > 0
        has_hi = rb < NBLK - 1

        def addrow(i, _):
            own = yloc[pl.ds(BLK_W + i * 16, 16)]
            va = jnp.where(has_lo, ybuf[pl.ds(i * 16, 16)], zeros16)
            vb = jnp.where(has_hi, ybuf[pl.ds(BLK_W + i * 16, 16)], zeros16)
            yout[pl.ds(i * 16, 16)] = own + va + vb
            return 0
        lax.fori_loop(0, BLK_W // 16, addrow, 0)

        pltpu.sync_copy(yout, out_hbm.at[b].at[pl.ds(start * NCP, BLK_W)])

    return k(featT, x2T, selp, param)


def kernel(x, conv1, conv2, seg_logits, conv_parameter, sel_idx):
    feat = jnp.concatenate([x, conv1, conv2], axis=1).reshape(B, K, N2)
    featT = jnp.transpose(feat, (0, 2, 1))
    featT = jnp.pad(featT, ((0, 0), (HALO, HALO), (0, KP - K)))
    featT = featT.reshape(B, (N2 + 2 * HALO) * KP)
    x2T = jnp.transpose(seg_logits.reshape(B, NC, N2), (0, 2, 1))
    x2T = jnp.pad(x2T, ((0, 0), (0, 0), (0, NCP - NC)))
    x2T = x2T.reshape(B, N2 * NCP)
    selp = jnp.concatenate(
        [sel_idx, jnp.tile(sel_idx[:, :1], (1, NSELP - NSEL))], axis=1)
    param = jnp.pad(conv_parameter, (0, KP - K))
    out = _sc_affinity(featT, x2T, selp.astype(jnp.int32), param)
    return out.reshape(B, N2, NCP)[:, :, :NC]


# speedup vs baseline: 11.6922x; 1.1669x over previous
import functools
import jax
import jax.numpy as jnp
from jax import lax
from jax.experimental import pallas as pl
from jax.experimental.pallas import tpu as pltpu
from jax.experimental.pallas import tpu_sc as plsc

B = 4
N = 32
N2 = N * N
K = 131
KP = 144
NCH = KP // 16
NC = 21
NCP = 32
RAD = 3
NOFF = 49
NSEL = 307
NSELP = 320
HALO = 128
ROWS = 128 + 2 * HALO
NBLK = 8
FT_W = ROWS * KP
SLAB_W = ROWS * NCP
BLK_W = 128 * NCP


def _sc_affinity(featT, x2T, selp, param):
    mesh = plsc.VectorSubcoreMesh(core_axis_name="c", subcore_axis_name="s")

    @functools.partial(
        pl.kernel,
        out_type=jax.ShapeDtypeStruct((B, N2 * NCP), jnp.float32),
        mesh=mesh,
        scratch_types=[
            pltpu.VMEM((FT_W,), jnp.float32),
            pltpu.VMEM((128 * NCP,), jnp.float32),
            pltpu.VMEM((NSELP,), jnp.int32),
            pltpu.SMEM((512,), jnp.float32),
            pltpu.SMEM((64,), jnp.int32),
            pltpu.VMEM((KP,), jnp.float32),
            pltpu.VMEM((NOFF * KP,), jnp.float32),
            pltpu.VMEM((KP,), jnp.float32),
            pltpu.VMEM((SLAB_W,), jnp.float32),
            pltpu.VMEM((2 * BLK_W,), jnp.float32),
            pltpu.VMEM((BLK_W,), jnp.float32),
            pltpu.VMEM_SHARED((16 * SLAB_W,), jnp.float32),
        ],
    )
    def k(featT_hbm, x2T_hbm, selp_hbm, param_hbm, out_hbm,
          ft, x2v, selv, msmem, lsm, pv, ev, rsv, yloc, ybuf, yout, ysh):
        sid = lax.axis_index("s")
        b_loc = jnp.remainder(sid, 2)
        b = lax.axis_index("c") * 2 + b_loc
        rb = sid // 2
        start = rb * 128

        zeros16 = jnp.zeros((16,), jnp.float32)

        pltpu.sync_copy(featT_hbm.at[b].at[pl.ds(start * KP, FT_W)], ft)
        pltpu.sync_copy(x2T_hbm.at[b].at[pl.ds(start * NCP, 128 * NCP)], x2v)
        pltpu.sync_copy(selp_hbm.at[b], selv)
        pltpu.sync_copy(param_hbm, pv)

        def zrow(i, _):
            yloc[pl.ds(i * 16, 16)] = zeros16
            return 0
        lax.fori_loop(0, SLAB_W // 16, zrow, 0)

        def mzero(i, _):
            msmem[i] = 0.0
            return 0
        lax.fori_loop(0, ROWS, mzero, 0)
        for j in range(NSELP // 16):
            idxv = selv[pl.ds(j * 16, 16)]
            for t in range(16):
                l = idxv[t] - start + HALO
                ok = (l >= 0) & (l < ROWS)
                msmem[jnp.where(ok, l, ROWS + 32)] = 1.0

        def pix(q_local, _):
            q = start + q_local
            mq = msmem[q_local + HALO]

            @pl.when(mq > 0.0)
            def _():
                lrow = q_local + HALO
                fq = [ft[pl.ds(lrow * KP + j * 16, 16)] for j in range(NCH)]
                r = q // N
                c = jnp.remainder(q, N)

                for j in range(NCH):
                    rsv[pl.ds(j * 16, 16)] = zeros16
                lsm[0] = 0

                def rowA(dri, _c):
                    rr = r + dri - RAD
                    row_ok = (rr >= 0) & (rr < N)
                    l2row = q_local + HALO + (dri - RAD) * N - RAD

                    def colA(dci, _c2):
                        cc = c + dci - RAD
                        l2 = l2row + dci
                        ok = row_ok & (cc >= 0) & (cc < N)
                        msk = jnp.where(ok, msmem[l2], 0.0)

                        @pl.when(msk > 0.0)
                        def _():
                            slot = lsm[0]
                            lsm[8 + slot] = l2
                            for j in range(NCH):
                                fn = ft[pl.ds(l2 * KP + j * 16, 16)]
                                e = jnp.exp(-jnp.abs(fq[j] - fn))
                                ev[pl.ds(slot * KP + j * 16, 16)] = e
                                rsv[pl.ds(j * 16, 16)] = (
                                    rsv[pl.ds(j * 16, 16)] + e)
                            lsm[0] = slot + 1
                        return _c2

                    lax.fori_loop(0, 2 * RAD + 1, colA, 0)
                    return _c

                lax.fori_loop(0, 2 * RAD + 1, rowA, 0)

                rs = [rsv[pl.ds(j * 16, 16)] for j in range(NCH)]
                inv = [pv[pl.ds(j * 16, 16)]
                       / jnp.where(rs[j] > 0.0, rs[j], 1.0)
                       for j in range(NCH)]
                x2a = x2v[pl.ds(q_local * NCP, 16)]
                x2b = x2v[pl.ds(q_local * NCP + 16, 16)]

                def offB(i, _c):
                    l2 = lsm[8 + i]
                    acc = ev[pl.ds(i * KP, 16)] * inv[0]
                    for j in range(1, NCH):
                        acc = acc + ev[pl.ds(i * KP + j * 16, 16)] * inv[j]
                    lanes = [acc[t] for t in range(16)]
                    while len(lanes) > 1:
                        lanes = [a2 + b2 for a2, b2 in
                                 zip(lanes[::2], lanes[1::2])]
                    w = lanes[0]
                    ya = yloc[pl.ds(l2 * NCP, 16)]
                    yb = yloc[pl.ds(l2 * NCP + 16, 16)]
                    yloc[pl.ds(l2 * NCP, 16)] = ya + w * x2a
                    yloc[pl.ds(l2 * NCP + 16, 16)] = yb + w * x2b
                    return _c

                lax.fori_loop(0, lsm[0], offB, 0)
            return 0

        lax.fori_loop(0, 128, pix, 0)

        pltpu.sync_copy(yloc, ysh.at[pl.ds(sid * SLAB_W, SLAB_W)])
        plsc.subcore_barrier()

        lo = jnp.maximum(sid - 2, 0)
        hi = jnp.minimum(sid + 2, 15)
        pltpu.sync_copy(ysh.at[pl.ds(lo * SLAB_W + 2 * BLK_W, BLK_W)],
                        ybuf.at[pl.ds(0, BLK_W)])
        pltpu.sync_copy(ysh.at[pl.ds(hi * SLAB_W, BLK_W)],
                        ybuf.at[pl.ds(BLK_W, BLK_W)])
        has_lo = rb > 0
        has_hi = rb < NBLK - 1

        def addrow(i, _):
            own = yloc[pl.ds(BLK_W + i * 16, 16)]
            va = jnp.where(has_lo, ybuf[pl.ds(i * 16, 16)], zeros16)
            vb = jnp.where(has_hi, ybuf[pl.ds(BLK_W + i * 16, 16)], zeros16)
            yout[pl.ds(i * 16, 16)] = own + va + vb
            return 0
        lax.fori_loop(0, BLK_W // 16, addrow, 0)

        pltpu.sync_copy(yout, out_hbm.at[b].at[pl.ds(start * NCP, BLK_W)])

    return k(featT, x2T, selp, param)


def kernel(x, conv1, conv2, seg_logits, conv_parameter, sel_idx):
    feat = jnp.concatenate([x, conv1, conv2], axis=1).reshape(B, K, N2)
    featT = jnp.transpose(feat, (0, 2, 1))
    featT = jnp.pad(featT, ((0, 0), (HALO, HALO), (0, KP - K)))
    featT = featT.reshape(B, (N2 + 2 * HALO) * KP)
    x2T = jnp.transpose(seg_logits.reshape(B, NC, N2), (0, 2, 1))
    x2T = jnp.pad(x2T, ((0, 0), (0, 0), (0, NCP - NC)))
    x2T = x2T.reshape(B, N2 * NCP)
    selp = jnp.concatenate(
        [sel_idx, jnp.tile(sel_idx[:, :1], (1, NSELP - NSEL))], axis=1)
    param = jnp.pad(conv_parameter, (0, KP - K))
    out = _sc_affinity(featT, x2T, selp.astype(jnp.int32), param)
    return out.reshape(B, N2, NCP)[:, :, :NC]


# speedup vs baseline: 13.5848x; 1.1619x over previous
import functools
import jax
import jax.numpy as jnp
from jax import lax
from jax.experimental import pallas as pl
from jax.experimental.pallas import tpu as pltpu
from jax.experimental.pallas import tpu_sc as plsc

B = 4
N = 32
N2 = N * N
K = 131
KP = 144
NCH = KP // 16
NC = 21
NCP = 32
RAD = 3
NOFF = 49
NSEL = 307
NSELP = 320
HALO = 128
ROWS = 128 + 2 * HALO
NBLK = 8
FT_W = ROWS * KP
SLAB_W = ROWS * NCP
BLK_W = 128 * NCP


def _sc_affinity(featT, x2T, selp, param):
    mesh = plsc.VectorSubcoreMesh(core_axis_name="c", subcore_axis_name="s")

    @functools.partial(
        pl.kernel,
        out_type=jax.ShapeDtypeStruct((B, N2 * NCP), jnp.float32),
        mesh=mesh,
        scratch_types=[
            pltpu.VMEM((FT_W,), jnp.float32),
            pltpu.VMEM((128 * NCP,), jnp.float32),
            pltpu.VMEM((NSELP,), jnp.int32),
            pltpu.SMEM((512,), jnp.float32),
            pltpu.SMEM((64,), jnp.int32),
            pltpu.VMEM((KP,), jnp.float32),
            pltpu.VMEM((NOFF * KP,), jnp.float32),
            pltpu.VMEM((KP,), jnp.float32),
            pltpu.VMEM((SLAB_W,), jnp.float32),
            pltpu.VMEM((2 * BLK_W,), jnp.float32),
            pltpu.VMEM((BLK_W,), jnp.float32),
            pltpu.VMEM_SHARED((16 * SLAB_W,), jnp.float32),
        ],
    )
    def k(featT_hbm, x2T_hbm, selp_hbm, param_hbm, out_hbm,
          ft, x2v, selv, msmem, lsm, pv, ev, rsv, yloc, ybuf, yout, ysh):
        sid = lax.axis_index("s")
        b_loc = jnp.remainder(sid, 2)
        b = lax.axis_index("c") * 2 + b_loc
        rb = sid // 2
        start = rb * 128

        zeros16 = jnp.zeros((16,), jnp.float32)

        pltpu.sync_copy(featT_hbm.at[b].at[pl.ds(start * KP, FT_W)], ft)
        pltpu.sync_copy(x2T_hbm.at[b].at[pl.ds(start * NCP, 128 * NCP)], x2v)
        pltpu.sync_copy(selp_hbm.at[b], selv)
        pltpu.sync_copy(param_hbm, pv)

        def zrow(i, _):
            for u in range(8):
                yloc[pl.ds(i * 128 + u * 16, 16)] = zeros16
            return 0
        lax.fori_loop(0, SLAB_W // 128, zrow, 0)

        def mzero(i, _):
            for u in range(8):
                msmem[i * 8 + u] = 0.0
            return 0
        lax.fori_loop(0, ROWS // 8, mzero, 0)
        for j in range(NSELP // 16):
            idxv = selv[pl.ds(j * 16, 16)]
            for t in range(16):
                l = idxv[t] - start + HALO
                ok = (l >= 0) & (l < ROWS)
                msmem[jnp.where(ok, l, ROWS + 32)] = 1.0

        def pix(q_local, _):
            q = start + q_local
            mq = msmem[q_local + HALO]

            @pl.when(mq > 0.0)
            def _():
                lrow = q_local + HALO
                fq = [ft[pl.ds(lrow * KP + j * 16, 16)] for j in range(NCH)]
                r = q // N
                c = jnp.remainder(q, N)

                for j in range(NCH):
                    rsv[pl.ds(j * 16, 16)] = zeros16
                lsm[0] = 0

                def rowA(dri, _c):
                    rr = r + dri - RAD
                    row_ok = (rr >= 0) & (rr < N)
                    l2row = q_local + HALO + (dri - RAD) * N - RAD

                    def colA(dci, _c2):
                        cc = c + dci - RAD
                        l2 = l2row + dci
                        ok = row_ok & (cc >= 0) & (cc < N)
                        msk = jnp.where(ok, msmem[l2], 0.0)

                        @pl.when(msk > 0.0)
                        def _():
                            slot = lsm[0]
                            lsm[8 + slot] = l2
                            for j in range(NCH):
                                fn = ft[pl.ds(l2 * KP + j * 16, 16)]
                                e = jnp.exp(-jnp.abs(fq[j] - fn))
                                ev[pl.ds(slot * KP + j * 16, 16)] = e
                                plsc.addupdate(rsv.at[pl.ds(j * 16, 16)], e)
                            lsm[0] = slot + 1
                        return _c2

                    lax.fori_loop(0, 2 * RAD + 1, colA, 0)
                    return _c

                lax.fori_loop(0, 2 * RAD + 1, rowA, 0)

                inv = [pv[pl.ds(j * 16, 16)] / rsv[pl.ds(j * 16, 16)]
                       for j in range(NCH)]
                x2a = x2v[pl.ds(q_local * NCP, 16)]
                x2b = x2v[pl.ds(q_local * NCP + 16, 16)]

                def offB(i, _c):
                    l2 = lsm[8 + i]
                    acc = ev[pl.ds(i * KP, 16)] * inv[0]
                    for j in range(1, NCH):
                        acc = acc + ev[pl.ds(i * KP + j * 16, 16)] * inv[j]
                    lanes = [acc[t] for t in range(16)]
                    while len(lanes) > 1:
                        lanes = [a2 + b2 for a2, b2 in
                                 zip(lanes[::2], lanes[1::2])]
                    w = lanes[0]
                    plsc.addupdate(yloc.at[pl.ds(l2 * NCP, 16)], w * x2a)
                    plsc.addupdate(yloc.at[pl.ds(l2 * NCP + 16, 16)], w * x2b)
                    return _c

                lax.fori_loop(0, lsm[0], offB, 0)
            return 0

        lax.fori_loop(0, 128, pix, 0)

        pltpu.sync_copy(yloc, ysh.at[pl.ds(sid * SLAB_W, SLAB_W)])
        plsc.subcore_barrier()

        lo = jnp.maximum(sid - 2, 0)
        hi = jnp.minimum(sid + 2, 15)
        pltpu.sync_copy(ysh.at[pl.ds(lo * SLAB_W + 2 * BLK_W, BLK_W)],
                        ybuf.at[pl.ds(0, BLK_W)])
        pltpu.sync_copy(ysh.at[pl.ds(hi * SLAB_W, BLK_W)],
                        ybuf.at[pl.ds(BLK_W, BLK_W)])
        has_lo = rb > 0
        has_hi = rb < NBLK - 1

        def addrow(i, _):
            for u in range(4):
                p = i * 64 + u * 16
                own = yloc[pl.ds(BLK_W + p, 16)]
                va = jnp.where(has_lo, ybuf[pl.ds(p, 16)], zeros16)
                vb = jnp.where(has_hi, ybuf[pl.ds(BLK_W + p, 16)], zeros16)
                yout[pl.ds(p, 16)] = own + va + vb
            return 0
        lax.fori_loop(0, BLK_W // 64, addrow, 0)

        pltpu.sync_copy(yout, out_hbm.at[b].at[pl.ds(start * NCP, BLK_W)])

    return k(featT, x2T, selp, param)


def kernel(x, conv1, conv2, seg_logits, conv_parameter, sel_idx):
    feat = jnp.concatenate([x, conv1, conv2], axis=1).reshape(B, K, N2)
    featT = jnp.transpose(feat, (0, 2, 1))
    featT = jnp.pad(featT, ((0, 0), (HALO, HALO), (0, KP - K)))
    featT = featT.reshape(B, (N2 + 2 * HALO) * KP)
    x2T = jnp.transpose(seg_logits.reshape(B, NC, N2), (0, 2, 1))
    x2T = jnp.pad(x2T, ((0, 0), (0, 0), (0, NCP - NC)))
    x2T = x2T.reshape(B, N2 * NCP)
    selp = jnp.concatenate(
        [sel_idx, jnp.tile(sel_idx[:, :1], (1, NSELP - NSEL))], axis=1)
    param = jnp.pad(conv_parameter, (0, KP - K))
    out = _sc_affinity(featT, x2T, selp.astype(jnp.int32), param)
    return out.reshape(B, N2, NCP)[:, :, :NC]


# speedup vs baseline: 21.5366x; 1.5854x over previous
import functools
import jax
import jax.numpy as jnp
from jax import lax
from jax.experimental import pallas as pl
from jax.experimental.pallas import tpu as pltpu
from jax.experimental.pallas import tpu_sc as plsc

B = 4
N = 32
N2 = N * N
K = 131
KP = 144
NCH = KP // 16
NC = 21
NCP = 32
RAD = 3
NOFF = 49
NSEL = 307
NSELP = 320
HALO = 128
ROWS = 128 + 2 * HALO
NBLK = 8
FT_W = ROWS * KP
SLAB_W = ROWS * NCP
BLK_W = 128 * NCP


def _sc_affinity(featT, x2T, selp, param):
    mesh = plsc.VectorSubcoreMesh(core_axis_name="c", subcore_axis_name="s")

    @functools.partial(
        pl.kernel,
        out_type=jax.ShapeDtypeStruct((B, N2 * NCP), jnp.float32),
        mesh=mesh,
        scratch_types=[
            pltpu.VMEM((FT_W,), jnp.float32),
            pltpu.VMEM((128 * NCP,), jnp.float32),
            pltpu.VMEM((NSELP,), jnp.int32),
            pltpu.SMEM((512,), jnp.float32),
            pltpu.SMEM((64,), jnp.int32),
            pltpu.VMEM((KP,), jnp.float32),
            pltpu.VMEM((NOFF * KP,), jnp.float32),
            pltpu.VMEM((KP,), jnp.float32),
            pltpu.VMEM((SLAB_W,), jnp.float32),
            pltpu.VMEM((2 * BLK_W,), jnp.float32),
            pltpu.VMEM((BLK_W,), jnp.float32),
            pltpu.VMEM_SHARED((16 * SLAB_W,), jnp.float32),
        ],
    )
    def k(featT_hbm, x2T_hbm, selp_hbm, param_hbm, out_hbm,
          ft, x2v, selv, msmem, lsm, pv, ev, rsv, yloc, ybuf, yout, ysh):
        sid = lax.axis_index("s")
        b_loc = jnp.remainder(sid, 2)
        b = lax.axis_index("c") * 2 + b_loc
        rb = sid // 2
        start = rb * 128

        zeros16 = jnp.zeros((16,), jnp.float32)

        pltpu.sync_copy(featT_hbm.at[b].at[pl.ds(start * KP, FT_W)], ft)
        pltpu.sync_copy(x2T_hbm.at[b].at[pl.ds(start * NCP, 128 * NCP)], x2v)
        pltpu.sync_copy(selp_hbm.at[b], selv)
        pltpu.sync_copy(param_hbm, pv)

        def zrow(i, _):
            for u in range(8):
                yloc[pl.ds(i * 128 + u * 16, 16)] = zeros16
            return 0
        lax.fori_loop(0, SLAB_W // 128, zrow, 0)

        def mzero(i, _):
            for u in range(8):
                msmem[i * 8 + u] = 0.0
            return 0
        lax.fori_loop(0, ROWS // 8, mzero, 0)
        for j in range(NSELP // 16):
            idxv = selv[pl.ds(j * 16, 16)]
            for t in range(16):
                l = idxv[t] - start + HALO
                ok = (l >= 0) & (l < ROWS)
                msmem[jnp.where(ok, l, ROWS + 32)] = 1.0

        def pix(q_local, _):
            q = start + q_local
            mq = msmem[q_local + HALO]

            @pl.when(mq > 0.0)
            def _():
                lrow = q_local + HALO
                fq = [ft[pl.ds(lrow * KP + j * 16, 16)] for j in range(NCH)]
                r = q // N
                c = jnp.remainder(q, N)

                for j in range(NCH):
                    rsv[pl.ds(j * 16, 16)] = zeros16
                lsm[0] = 0

                def rowA(dri, _c):
                    rr = r + dri - RAD
                    row_ok = (rr >= 0) & (rr < N)
                    l2row = q_local + HALO + (dri - RAD) * N - RAD

                    def colA(dci, _c2):
                        cc = c + dci - RAD
                        l2 = l2row + dci
                        ok = row_ok & (cc >= 0) & (cc < N)
                        msk = jnp.where(ok, msmem[l2], 0.0)

                        @pl.when(msk > 0.0)
                        def _():
                            slot = lsm[0]
                            lsm[8 + slot] = l2
                            fns = [ft[pl.ds(l2 * KP + j * 16, 16)]
                                   for j in range(NCH)]
                            es = [jnp.exp(-jnp.abs(fq[j] - fns[j]))
                                  for j in range(NCH)]
                            for j in range(NCH):
                                ev[pl.ds(slot * KP + j * 16, 16)] = es[j]
                            for j in range(NCH):
                                plsc.addupdate(rsv.at[pl.ds(j * 16, 16)],
                                               es[j])
                            lsm[0] = slot + 1
                        return _c2

                    lax.fori_loop(0, 2 * RAD + 1, colA, 0)
                    return _c

                lax.fori_loop(0, 2 * RAD + 1, rowA, 0)

                inv = [pv[pl.ds(j * 16, 16)] / rsv[pl.ds(j * 16, 16)]
                       for j in range(NCH)]
                x2a = x2v[pl.ds(q_local * NCP, 16)]
                x2b = x2v[pl.ds(q_local * NCP + 16, 16)]

                def offB(i, _c):
                    l2 = lsm[8 + i]
                    evs = [ev[pl.ds(i * KP + j * 16, 16)]
                           for j in range(NCH)]
                    prods = [evs[j] * inv[j] for j in range(NCH)]
                    while len(prods) > 1:
                        prods = ([a2 + b2 for a2, b2 in
                                  zip(prods[::2], prods[1::2])]
                                 + ([prods[-1]] if len(prods) % 2 else []))
                    acc = prods[0]
                    lanes = [acc[t] for t in range(16)]
                    while len(lanes) > 1:
                        lanes = [a2 + b2 for a2, b2 in
                                 zip(lanes[::2], lanes[1::2])]
                    w = lanes[0]
                    plsc.addupdate(yloc.at[pl.ds(l2 * NCP, 16)], w * x2a)
                    plsc.addupdate(yloc.at[pl.ds(l2 * NCP + 16, 16)], w * x2b)
                    return _c

                lax.fori_loop(0, lsm[0], offB, 0)
            return 0

        lax.fori_loop(0, 128, pix, 0)

        pltpu.sync_copy(yloc, ysh.at[pl.ds(sid * SLAB_W, SLAB_W)])
        plsc.subcore_barrier()

        lo = jnp.maximum(sid - 2, 0)
        hi = jnp.minimum(sid + 2, 15)
        pltpu.sync_copy(ysh.at[pl.ds(lo * SLAB_W + 2 * BLK_W, BLK_W)],
                        ybuf.at[pl.ds(0, BLK_W)])
        pltpu.sync_copy(ysh.at[pl.ds(hi * SLAB_W, BLK_W)],
                        ybuf.at[pl.ds(BLK_W, BLK_W)])
        has_lo = rb > 0
        has_hi = rb < NBLK - 1

        def addrow(i, _):
            for u in range(4):
                p = i * 64 + u * 16
                own = yloc[pl.ds(BLK_W + p, 16)]
                va = jnp.where(has_lo, ybuf[pl.ds(p, 16)], zeros16)
                vb = jnp.where(has_hi, ybuf[pl.ds(BLK_W + p, 16)], zeros16)
                yout[pl.ds(p, 16)] = own + va + vb
            return 0
        lax.fori_loop(0, BLK_W // 64, addrow, 0)

        pltpu.sync_copy(yout, out_hbm.at[b].at[pl.ds(start * NCP, BLK_W)])

    return k(featT, x2T, selp, param)


def kernel(x, conv1, conv2, seg_logits, conv_parameter, sel_idx):
    feat = jnp.concatenate([x, conv1, conv2], axis=1).reshape(B, K, N2)
    featT = jnp.transpose(feat, (0, 2, 1))
    featT = jnp.pad(featT, ((0, 0), (HALO, HALO), (0, KP - K)))
    featT = featT.reshape(B, (N2 + 2 * HALO) * KP)
    x2T = jnp.transpose(seg_logits.reshape(B, NC, N2), (0, 2, 1))
    x2T = jnp.pad(x2T, ((0, 0), (0, 0), (0, NCP - NC)))
    x2T = x2T.reshape(B, N2 * NCP)
    selp = jnp.concatenate(
        [sel_idx, jnp.tile(sel_idx[:, :1], (1, NSELP - NSEL))], axis=1)
    param = jnp.pad(conv_parameter, (0, KP - K))
    out = _sc_affinity(featT, x2T, selp.astype(jnp.int32), param)
    return out.reshape(B, N2, NCP)[:, :, :NC]


# speedup vs baseline: 23.5128x; 1.0918x over previous
import functools
import jax
import jax.numpy as jnp
from jax import lax
from jax.experimental import pallas as pl
from jax.experimental.pallas import tpu as pltpu
from jax.experimental.pallas import tpu_sc as plsc

B = 4
N = 32
N2 = N * N
K = 131
KP = 144
NCH = KP // 16
NC = 21
NCP = 32
RAD = 3
NOFF = 49
NSEL = 307
NSELP = 320
HALO = 128
ROWS = 128 + 2 * HALO
NBLK = 8
FT_W = ROWS * KP
SLAB_W = ROWS * NCP
BLK_W = 128 * NCP


def _sc_affinity(featT, x2T, selp, param):
    mesh = plsc.VectorSubcoreMesh(core_axis_name="c", subcore_axis_name="s")

    @functools.partial(
        pl.kernel,
        out_type=jax.ShapeDtypeStruct((B, N2 * NCP), jnp.float32),
        mesh=mesh,
        scratch_types=[
            pltpu.VMEM((FT_W,), jnp.float32),
            pltpu.VMEM((128 * NCP,), jnp.float32),
            pltpu.VMEM((NSELP,), jnp.int32),
            pltpu.SMEM((512,), jnp.float32),
            pltpu.SMEM((64,), jnp.int32),
            pltpu.VMEM((KP,), jnp.float32),
            pltpu.VMEM((NOFF * KP,), jnp.float32),
            pltpu.VMEM((KP,), jnp.float32),
            pltpu.VMEM((SLAB_W,), jnp.float32),
            pltpu.VMEM((2 * BLK_W,), jnp.float32),
            pltpu.VMEM((BLK_W,), jnp.float32),
            pltpu.VMEM_SHARED((16 * SLAB_W,), jnp.float32),
        ],
    )
    def k(featT_hbm, x2T_hbm, selp_hbm, param_hbm, out_hbm,
          ft, x2v, selv, msmem, lsm, pv, ev, rsv, yloc, ybuf, yout, ysh):
        sid = lax.axis_index("s")
        b_loc = jnp.remainder(sid, 2)
        b = lax.axis_index("c") * 2 + b_loc
        rb = sid // 2
        start = rb * 128

        zeros16 = jnp.zeros((16,), jnp.float32)
        lane = lax.iota(jnp.int32, 16)
        perms = [(lane ^ s)[:, None] for s in (8, 4, 2, 1)]
        gdn = lax.GatherDimensionNumbers(
            offset_dims=(), collapsed_slice_dims=(0,), start_index_map=(0,))

        def lane_perm(v, p):
            return lax.gather(v, p, gdn, (1,),
                              mode=lax.GatherScatterMode.PROMISE_IN_BOUNDS)

        pltpu.sync_copy(featT_hbm.at[b].at[pl.ds(start * KP, FT_W)], ft)
        pltpu.sync_copy(x2T_hbm.at[b].at[pl.ds(start * NCP, 128 * NCP)], x2v)
        pltpu.sync_copy(selp_hbm.at[b], selv)
        pltpu.sync_copy(param_hbm, pv)

        def zrow(i, _):
            for u in range(8):
                yloc[pl.ds(i * 128 + u * 16, 16)] = zeros16
            return 0
        lax.fori_loop(0, SLAB_W // 128, zrow, 0)

        def mzero(i, _):
            for u in range(8):
                msmem[i * 8 + u] = 0.0
            return 0
        lax.fori_loop(0, ROWS // 8, mzero, 0)
        for j in range(NSELP // 16):
            idxv = selv[pl.ds(j * 16, 16)]
            for t in range(16):
                l = idxv[t] - start + HALO
                ok = (l >= 0) & (l < ROWS)
                msmem[jnp.where(ok, l, ROWS + 32)] = 1.0

        def pix(q_local, _):
            q = start + q_local
            mq = msmem[q_local + HALO]

            @pl.when(mq > 0.0)
            def _():
                lrow = q_local + HALO
                fq = [ft[pl.ds(lrow * KP + j * 16, 16)] for j in range(NCH)]
                r = q // N
                c = jnp.remainder(q, N)

                for j in range(NCH):
                    rsv[pl.ds(j * 16, 16)] = zeros16
                lsm[0] = 0

                def rowA(dri, _c):
                    rr = r + dri - RAD
                    row_ok = (rr >= 0) & (rr < N)
                    l2row = q_local + HALO + (dri - RAD) * N - RAD

                    def colA(dci, _c2):
                        cc = c + dci - RAD
                        l2 = l2row + dci
                        ok = row_ok & (cc >= 0) & (cc < N)
                        msk = jnp.where(ok, msmem[l2], 0.0)

                        @pl.when(msk > 0.0)
                        def _():
                            slot = lsm[0]
                            lsm[8 + slot] = l2
                            fns = [ft[pl.ds(l2 * KP + j * 16, 16)]
                                   for j in range(NCH)]
                            es = [jnp.exp(-jnp.abs(fq[j] - fns[j]))
                                  for j in range(NCH)]
                            for j in range(NCH):
                                ev[pl.ds(slot * KP + j * 16, 16)] = es[j]
                            for j in range(NCH):
                                plsc.addupdate(rsv.at[pl.ds(j * 16, 16)],
                                               es[j])
                            lsm[0] = slot + 1
                        return _c2

                    lax.fori_loop(0, 2 * RAD + 1, colA, 0)
                    return _c

                lax.fori_loop(0, 2 * RAD + 1, rowA, 0)

                inv = [pv[pl.ds(j * 16, 16)] / rsv[pl.ds(j * 16, 16)]
                       for j in range(NCH)]
                x2a = x2v[pl.ds(q_local * NCP, 16)]
                x2b = x2v[pl.ds(q_local * NCP + 16, 16)]

                def offB(i, _c):
                    l2 = lsm[8 + i]
                    evs = [ev[pl.ds(i * KP + j * 16, 16)]
                           for j in range(NCH)]
                    prods = [evs[j] * inv[j] for j in range(NCH)]
                    while len(prods) > 1:
                        prods = ([a2 + b2 for a2, b2 in
                                  zip(prods[::2], prods[1::2])]
                                 + ([prods[-1]] if len(prods) % 2 else []))
                    acc = prods[0]
                    for p in perms:
                        acc = acc + lane_perm(acc, p)
                    plsc.addupdate(yloc.at[pl.ds(l2 * NCP, 16)], acc * x2a)
                    plsc.addupdate(yloc.at[pl.ds(l2 * NCP + 16, 16)], acc * x2b)
                    return _c

                lax.fori_loop(0, lsm[0], offB, 0)
            return 0

        lax.fori_loop(0, 128, pix, 0)

        pltpu.sync_copy(yloc, ysh.at[pl.ds(sid * SLAB_W, SLAB_W)])
        plsc.subcore_barrier()

        lo = jnp.maximum(sid - 2, 0)
        hi = jnp.minimum(sid + 2, 15)
        pltpu.sync_copy(ysh.at[pl.ds(lo * SLAB_W + 2 * BLK_W, BLK_W)],
                        ybuf.at[pl.ds(0, BLK_W)])
        pltpu.sync_copy(ysh.at[pl.ds(hi * SLAB_W, BLK_W)],
                        ybuf.at[pl.ds(BLK_W, BLK_W)])
        has_lo = rb > 0
        has_hi = rb < NBLK - 1

        def addrow(i, _):
            for u in range(4):
                p = i * 64 + u * 16
                own = yloc[pl.ds(BLK_W + p, 16)]
                va = jnp.where(has_lo, ybuf[pl.ds(p, 16)], zeros16)
                vb = jnp.where(has_hi, ybuf[pl.ds(BLK_W + p, 16)], zeros16)
                yout[pl.ds(p, 16)] = own + va + vb
            return 0
        lax.fori_loop(0, BLK_W // 64, addrow, 0)

        pltpu.sync_copy(yout, out_hbm.at[b].at[pl.ds(start * NCP, BLK_W)])

    return k(featT, x2T, selp, param)


def kernel(x, conv1, conv2, seg_logits, conv_parameter, sel_idx):
    feat = jnp.concatenate([x, conv1, conv2], axis=1).reshape(B, K, N2)
    featT = jnp.transpose(feat, (0, 2, 1))
    featT = jnp.pad(featT, ((0, 0), (HALO, HALO), (0, KP - K)))
    featT = featT.reshape(B, (N2 + 2 * HALO) * KP)
    x2T = jnp.transpose(seg_logits.reshape(B, NC, N2), (0, 2, 1))
    x2T = jnp.pad(x2T, ((0, 0), (0, 0), (0, NCP - NC)))
    x2T = x2T.reshape(B, N2 * NCP)
    selp = jnp.concatenate(
        [sel_idx, jnp.tile(sel_idx[:, :1], (1, NSELP - NSEL))], axis=1)
    param = jnp.pad(conv_parameter, (0, KP - K))
    out = _sc_affinity(featT, x2T, selp.astype(jnp.int32), param)
    return out.reshape(B, N2, NCP)[:, :, :NC]


# speedup vs baseline: 28.0637x; 1.1935x over previous
import functools
import jax
import jax.numpy as jnp
from jax import lax
from jax.experimental import pallas as pl
from jax.experimental.pallas import tpu as pltpu
from jax.experimental.pallas import tpu_sc as plsc

B = 4
N = 32
N2 = N * N
K = 131
KP = 144
NCH = KP // 16
NC = 21
NCP = 32
RAD = 3
NOFF = 49
NSEL = 307
NSELP = 320
HALO = 128
ROWS = 128 + 2 * HALO
NBLK = 8
FT_W = ROWS * KP
SLAB_W = ROWS * NCP
BLK_W = 128 * NCP


def _sc_affinity(featT, x2T, selp, param):
    mesh = plsc.VectorSubcoreMesh(core_axis_name="c", subcore_axis_name="s")

    @functools.partial(
        pl.kernel,
        out_type=jax.ShapeDtypeStruct((B, N2 * NCP), jnp.float32),
        mesh=mesh,
        scratch_types=[
            pltpu.VMEM((FT_W,), jnp.float32),
            pltpu.VMEM((128 * NCP,), jnp.float32),
            pltpu.VMEM((NSELP,), jnp.int32),
            pltpu.SMEM((512,), jnp.float32),
            pltpu.SMEM((64,), jnp.int32),
            pltpu.VMEM((KP,), jnp.float32),
            pltpu.VMEM((NOFF * KP,), jnp.float32),
            pltpu.VMEM((KP,), jnp.float32),
            pltpu.VMEM((SLAB_W,), jnp.float32),
            pltpu.VMEM((2 * BLK_W,), jnp.float32),
            pltpu.VMEM((BLK_W,), jnp.float32),
            pltpu.VMEM_SHARED((16 * SLAB_W,), jnp.float32),
        ],
    )
    def k(featT_hbm, x2T_hbm, selp_hbm, param_hbm, out_hbm,
          ft, x2v, selv, msmem, lsm, pv, ev, rsv, yloc, ybuf, yout, ysh):
        sid = lax.axis_index("s")
        b_loc = jnp.remainder(sid, 2)
        b = lax.axis_index("c") * 2 + b_loc
        rb = sid // 2
        start = rb * 128

        zeros16 = jnp.zeros((16,), jnp.float32)
        lane = lax.iota(jnp.int32, 16)
        perms = [(lane ^ s)[:, None] for s in (8, 4, 2, 1)]
        gdn = lax.GatherDimensionNumbers(
            offset_dims=(), collapsed_slice_dims=(0,), start_index_map=(0,))

        def lane_perm(v, p):
            return lax.gather(v, p, gdn, (1,),
                              mode=lax.GatherScatterMode.PROMISE_IN_BOUNDS)

        pltpu.sync_copy(featT_hbm.at[b].at[pl.ds(start * KP, FT_W)], ft)
        pltpu.sync_copy(x2T_hbm.at[b].at[pl.ds(start * NCP, 128 * NCP)], x2v)
        pltpu.sync_copy(selp_hbm.at[b], selv)
        pltpu.sync_copy(param_hbm, pv)

        def zrow(i, _):
            for u in range(8):
                yloc[pl.ds(i * 128 + u * 16, 16)] = zeros16
            return 0
        lax.fori_loop(0, SLAB_W // 128, zrow, 0)

        def mzero(i, _):
            for u in range(8):
                msmem[i * 8 + u] = 0.0
            return 0
        lax.fori_loop(0, ROWS // 8, mzero, 0)
        for j in range(NSELP // 16):
            idxv = selv[pl.ds(j * 16, 16)]
            for t in range(16):
                l = idxv[t] - start + HALO
                ok = (l >= 0) & (l < ROWS)
                msmem[jnp.where(ok, l, ROWS + 32)] = 1.0

        def pix(q_local, _):
            q = start + q_local
            mq = msmem[q_local + HALO]

            @pl.when(mq > 0.0)
            def _():
                lrow = q_local + HALO
                fq = [ft[pl.ds(lrow * KP + j * 16, 16)] for j in range(NCH)]
                r = q // N
                c = jnp.remainder(q, N)

                for j in range(NCH):
                    rsv[pl.ds(j * 16, 16)] = zeros16

                def rowA(dri, cnt):
                    rr = r + dri - RAD
                    row_ok = (rr >= 0) & (rr < N)
                    l2row = q_local + HALO + (dri - RAD) * N - RAD
                    for dci in range(2 * RAD + 1):
                        cc = c + dci - RAD
                        l2 = l2row + dci
                        ok = row_ok & (cc >= 0) & (cc < N)
                        take = jnp.where(ok, msmem[l2], 0.0) > 0.0
                        lsm[jnp.where(take, 8 + cnt, 63)] = l2
                        cnt = cnt + jnp.where(take, 1, 0)
                    return cnt

                cnt = lax.fori_loop(0, 2 * RAD + 1, rowA, 0)
                lsm[0] = cnt

                def offA(i, _c):
                    l2 = lsm[8 + i]
                    fns = [ft[pl.ds(l2 * KP + j * 16, 16)]
                           for j in range(NCH)]
                    es = [jnp.exp(-jnp.abs(fq[j] - fns[j]))
                          for j in range(NCH)]
                    for j in range(NCH):
                        ev[pl.ds(i * KP + j * 16, 16)] = es[j]
                    for j in range(NCH):
                        plsc.addupdate(rsv.at[pl.ds(j * 16, 16)], es[j])
                    return _c

                lax.fori_loop(0, cnt, offA, 0)

                inv = [pv[pl.ds(j * 16, 16)] / rsv[pl.ds(j * 16, 16)]
                       for j in range(NCH)]
                x2a = x2v[pl.ds(q_local * NCP, 16)]
                x2b = x2v[pl.ds(q_local * NCP + 16, 16)]

                def offB(i, _c):
                    l2 = lsm[8 + i]
                    evs = [ev[pl.ds(i * KP + j * 16, 16)]
                           for j in range(NCH)]
                    prods = [evs[j] * inv[j] for j in range(NCH)]
                    while len(prods) > 1:
                        prods = ([a2 + b2 for a2, b2 in
                                  zip(prods[::2], prods[1::2])]
                                 + ([prods[-1]] if len(prods) % 2 else []))
                    acc = prods[0]
                    for p in perms:
                        acc = acc + lane_perm(acc, p)
                    plsc.addupdate(yloc.at[pl.ds(l2 * NCP, 16)], acc * x2a)
                    plsc.addupdate(yloc.at[pl.ds(l2 * NCP + 16, 16)], acc * x2b)
                    return _c

                lax.fori_loop(0, lsm[0], offB, 0)
            return 0

        lax.fori_loop(0, 128, pix, 0)

        pltpu.sync_copy(yloc, ysh.at[pl.ds(sid * SLAB_W, SLAB_W)])
        plsc.subcore_barrier()

        lo = jnp.maximum(sid - 2, 0)
        hi = jnp.minimum(sid + 2, 15)
        pltpu.sync_copy(ysh.at[pl.ds(lo * SLAB_W + 2 * BLK_W, BLK_W)],
                        ybuf.at[pl.ds(0, BLK_W)])
        pltpu.sync_copy(ysh.at[pl.ds(hi * SLAB_W, BLK_W)],
                        ybuf.at[pl.ds(BLK_W, BLK_W)])
        has_lo = rb > 0
        has_hi = rb < NBLK - 1

        def addrow(i, _):
            for u in range(4):
                p = i * 64 + u * 16
                own = yloc[pl.ds(BLK_W + p, 16)]
                va = jnp.where(has_lo, ybuf[pl.ds(p, 16)], zeros16)
                vb = jnp.where(has_hi, ybuf[pl.ds(BLK_W + p, 16)], zeros16)
                yout[pl.ds(p, 16)] = own + va + vb
            return 0
        lax.fori_loop(0, BLK_W // 64, addrow, 0)

        pltpu.sync_copy(yout, out_hbm.at[b].at[pl.ds(start * NCP, BLK_W)])

    return k(featT, x2T, selp, param)


def kernel(x, conv1, conv2, seg_logits, conv_parameter, sel_idx):
    feat = jnp.concatenate([x, conv1, conv2], axis=1).reshape(B, K, N2)
    featT = jnp.transpose(feat, (0, 2, 1))
    featT = jnp.pad(featT, ((0, 0), (HALO, HALO), (0, KP - K)))
    featT = featT.reshape(B, (N2 + 2 * HALO) * KP)
    x2T = jnp.transpose(seg_logits.reshape(B, NC, N2), (0, 2, 1))
    x2T = jnp.pad(x2T, ((0, 0), (0, 0), (0, NCP - NC)))
    x2T = x2T.reshape(B, N2 * NCP)
    selp = jnp.concatenate(
        [sel_idx, jnp.tile(sel_idx[:, :1], (1, NSELP - NSEL))], axis=1)
    param = jnp.pad(conv_parameter, (0, KP - K))
    out = _sc_affinity(featT, x2T, selp.astype(jnp.int32), param)
    return out.reshape(B, N2, NCP)[:, :, :NC]


# speedup vs baseline: 29.2629x; 1.0427x over previous
import functools
import jax
import jax.numpy as jnp
from jax import lax
from jax.experimental import pallas as pl
from jax.experimental.pallas import tpu as pltpu
from jax.experimental.pallas import tpu_sc as plsc

B = 4
N = 32
N2 = N * N
K = 131
KP = 144
NCH = KP // 16
NC = 21
NCP = 32
RAD = 3
NOFF = 49
NSEL = 307
NSELP = 320
HALO = 128
ROWS = 128 + 2 * HALO
NBLK = 8
FT_W = ROWS * KP
SLAB_W = ROWS * NCP
BLK_W = 128 * NCP


def _sc_affinity(featT, x2T, selp, param):
    mesh = plsc.VectorSubcoreMesh(core_axis_name="c", subcore_axis_name="s")

    @functools.partial(
        pl.kernel,
        out_type=jax.ShapeDtypeStruct((B, N2 * NCP), jnp.float32),
        mesh=mesh,
        scratch_types=[
            pltpu.VMEM((FT_W,), jnp.float32),
            pltpu.VMEM((128 * NCP,), jnp.float32),
            pltpu.VMEM((NSELP,), jnp.int32),
            pltpu.SMEM((512,), jnp.float32),
            pltpu.SMEM((64,), jnp.int32),
            pltpu.VMEM((KP,), jnp.float32),
            pltpu.VMEM((NOFF * KP,), jnp.float32),
            pltpu.VMEM((SLAB_W,), jnp.float32),
            pltpu.VMEM((2 * BLK_W,), jnp.float32),
            pltpu.VMEM((BLK_W,), jnp.float32),
            pltpu.VMEM_SHARED((16 * SLAB_W,), jnp.float32),
        ],
    )
    def k(featT_hbm, x2T_hbm, selp_hbm, param_hbm, out_hbm,
          ft, x2v, selv, msmem, lsm, pv, ev, yloc, ybuf, yout, ysh):
        sid = lax.axis_index("s")
        b_loc = jnp.remainder(sid, 2)
        b = lax.axis_index("c") * 2 + b_loc
        rb = sid // 2
        start = rb * 128

        zeros16 = jnp.zeros((16,), jnp.float32)
        lane = lax.iota(jnp.int32, 16)
        perms = [(lane ^ s)[:, None] for s in (8, 4, 2, 1)]
        gdn = lax.GatherDimensionNumbers(
            offset_dims=(), collapsed_slice_dims=(0,), start_index_map=(0,))

        def lane_perm(v, p):
            return lax.gather(v, p, gdn, (1,),
                              mode=lax.GatherScatterMode.PROMISE_IN_BOUNDS)

        pltpu.sync_copy(featT_hbm.at[b].at[pl.ds(start * KP, FT_W)], ft)
        pltpu.sync_copy(x2T_hbm.at[b].at[pl.ds(start * NCP, 128 * NCP)], x2v)
        pltpu.sync_copy(selp_hbm.at[b], selv)
        pltpu.sync_copy(param_hbm, pv)

        def zrow(i, _):
            for u in range(8):
                yloc[pl.ds(i * 128 + u * 16, 16)] = zeros16
            return 0
        lax.fori_loop(0, SLAB_W // 128, zrow, 0)

        def mzero(i, _):
            for u in range(8):
                msmem[i * 8 + u] = 0.0
            return 0
        lax.fori_loop(0, ROWS // 8, mzero, 0)
        for j in range(NSELP // 16):
            idxv = selv[pl.ds(j * 16, 16)]
            for t in range(16):
                l = idxv[t] - start + HALO
                ok = (l >= 0) & (l < ROWS)
                msmem[jnp.where(ok, l, ROWS + 32)] = 1.0

        def pix(q_local, _):
            q = start + q_local
            mq = msmem[q_local + HALO]

            @pl.when(mq > 0.0)
            def _():
                lrow = q_local + HALO
                fq = [ft[pl.ds(lrow * KP + j * 16, 16)] for j in range(NCH)]
                r = q // N
                c = jnp.remainder(q, N)

                def rowA(dri, cnt):
                    rr = r + dri - RAD
                    row_ok = (rr >= 0) & (rr < N)
                    l2row = q_local + HALO + (dri - RAD) * N - RAD
                    for dci in range(2 * RAD + 1):
                        cc = c + dci - RAD
                        l2 = l2row + dci
                        ok = row_ok & (cc >= 0) & (cc < N)
                        take = jnp.where(ok, msmem[l2], 0.0) > 0.0
                        lsm[jnp.where(take, 8 + cnt, 63)] = l2
                        cnt = cnt + jnp.where(take, 1, 0)
                    return cnt

                cnt = lax.fori_loop(0, 2 * RAD + 1, rowA, 0)
                lsm[0] = cnt

                def offA(i, rs):
                    l2 = lsm[8 + i]
                    fns = [ft[pl.ds(l2 * KP + j * 16, 16)]
                           for j in range(NCH)]
                    es = [jnp.exp(-jnp.abs(fq[j] - fns[j]))
                          for j in range(NCH)]
                    for j in range(NCH):
                        ev[pl.ds(i * KP + j * 16, 16)] = es[j]
                    return tuple(rs[j] + es[j] for j in range(NCH))

                rs = lax.fori_loop(0, cnt, offA,
                                   tuple(zeros16 for _ in range(NCH)))

                inv = [pv[pl.ds(j * 16, 16)] / rs[j] for j in range(NCH)]
                x2a = x2v[pl.ds(q_local * NCP, 16)]
                x2b = x2v[pl.ds(q_local * NCP + 16, 16)]

                def offB(i, _c):
                    l2 = lsm[8 + i]
                    evs = [ev[pl.ds(i * KP + j * 16, 16)]
                           for j in range(NCH)]
                    prods = [evs[j] * inv[j] for j in range(NCH)]
                    while len(prods) > 1:
                        prods = ([a2 + b2 for a2, b2 in
                                  zip(prods[::2], prods[1::2])]
                                 + ([prods[-1]] if len(prods) % 2 else []))
                    acc = prods[0]
                    for p in perms:
                        acc = acc + lane_perm(acc, p)
                    plsc.addupdate(yloc.at[pl.ds(l2 * NCP, 16)], acc * x2a)
                    plsc.addupdate(yloc.at[pl.ds(l2 * NCP + 16, 16)], acc * x2b)
                    return _c

                lax.fori_loop(0, lsm[0], offB, 0)
            return 0

        lax.fori_loop(0, 128, pix, 0)

        pltpu.sync_copy(yloc, ysh.at[pl.ds(sid * SLAB_W, SLAB_W)])
        plsc.subcore_barrier()

        lo = jnp.maximum(sid - 2, 0)
        hi = jnp.minimum(sid + 2, 15)
        pltpu.sync_copy(ysh.at[pl.ds(lo * SLAB_W + 2 * BLK_W, BLK_W)],
                        ybuf.at[pl.ds(0, BLK_W)])
        pltpu.sync_copy(ysh.at[pl.ds(hi * SLAB_W, BLK_W)],
                        ybuf.at[pl.ds(BLK_W, BLK_W)])
        has_lo = rb > 0
        has_hi = rb < NBLK - 1

        def addrow(i, _):
            for u in range(4):
                p = i * 64 + u * 16
                own = yloc[pl.ds(BLK_W + p, 16)]
                va = jnp.where(has_lo, ybuf[pl.ds(p, 16)], zeros16)
                vb = jnp.where(has_hi, ybuf[pl.ds(BLK_W + p, 16)], zeros16)
                yout[pl.ds(p, 16)] = own + va + vb
            return 0
        lax.fori_loop(0, BLK_W // 64, addrow, 0)

        pltpu.sync_copy(yout, out_hbm.at[b].at[pl.ds(start * NCP, BLK_W)])

    return k(featT, x2T, selp, param)


def kernel(x, conv1, conv2, seg_logits, conv_parameter, sel_idx):
    feat = jnp.concatenate([x, conv1, conv2], axis=1).reshape(B, K, N2)
    featT = jnp.transpose(feat, (0, 2, 1))
    featT = jnp.pad(featT, ((0, 0), (HALO, HALO), (0, KP - K)))
    featT = featT.reshape(B, (N2 + 2 * HALO) * KP)
    x2T = jnp.transpose(seg_logits.reshape(B, NC, N2), (0, 2, 1))
    x2T = jnp.pad(x2T, ((0, 0), (0, 0), (0, NCP - NC)))
    x2T = x2T.reshape(B, N2 * NCP)
    selp = jnp.concatenate(
        [sel_idx, jnp.tile(sel_idx[:, :1], (1, NSELP - NSEL))], axis=1)
    param = jnp.pad(conv_parameter, (0, KP - K))
    out = _sc_affinity(featT, x2T, selp.astype(jnp.int32), param)
    return out.reshape(B, N2, NCP)[:, :, :NC]


# speedup vs baseline: 30.1605x; 1.0307x over previous
import functools
import jax
import jax.numpy as jnp
from jax import lax
from jax.experimental import pallas as pl
from jax.experimental.pallas import tpu as pltpu
from jax.experimental.pallas import tpu_sc as plsc

B = 4
N = 32
N2 = N * N
K = 131
KP = 144
NCH = KP // 16
NC = 21
NCP = 32
RAD = 3
NOFF = 49
NSEL = 307
NSELP = 320
HALO = 128
ROWS = 128 + 2 * HALO
NBLK = 8
FT_W = ROWS * KP
SLAB_W = ROWS * NCP
BLK_W = 128 * NCP


def _sc_affinity(featT, x2T, selp, param):
    mesh = plsc.VectorSubcoreMesh(core_axis_name="c", subcore_axis_name="s")

    @functools.partial(
        pl.kernel,
        out_type=jax.ShapeDtypeStruct((B, N2 * NCP), jnp.float32),
        mesh=mesh,
        scratch_types=[
            pltpu.VMEM((FT_W,), jnp.float32),
            pltpu.VMEM((192 * NCP,), jnp.float32),
            pltpu.VMEM((NSELP,), jnp.int32),
            pltpu.SMEM((512,), jnp.float32),
            pltpu.SMEM((64,), jnp.int32),
            pltpu.VMEM((KP,), jnp.float32),
            pltpu.VMEM((NOFF * KP,), jnp.float32),
            pltpu.VMEM((SLAB_W,), jnp.float32),
            pltpu.VMEM((2 * BLK_W,), jnp.float32),
            pltpu.VMEM((BLK_W,), jnp.float32),
            pltpu.VMEM_SHARED((16 * SLAB_W,), jnp.float32),
        ],
    )
    def k(featT_hbm, x2T_hbm, selp_hbm, param_hbm, out_hbm,
          ft, x2v, selv, msmem, lsm, pv, ev, yloc, ybuf, yout, ysh):
        sid = lax.axis_index("s")
        b_loc = jnp.remainder(sid, 2)
        b = lax.axis_index("c") * 2 + b_loc
        rb = sid // 2
        start = rb * 128

        zeros16 = jnp.zeros((16,), jnp.float32)
        lane = lax.iota(jnp.int32, 16)
        perms = [(lane ^ s)[:, None] for s in (8, 4, 2, 1)]
        gdn = lax.GatherDimensionNumbers(
            offset_dims=(), collapsed_slice_dims=(0,), start_index_map=(0,))

        def lane_perm(v, p):
            return lax.gather(v, p, gdn, (1,),
                              mode=lax.GatherScatterMode.PROMISE_IN_BOUNDS)

        podd = jnp.remainder(rb, 2)
        x2base = start - podd * 64
        pltpu.sync_copy(featT_hbm.at[b].at[pl.ds(start * KP, FT_W)], ft)
        pltpu.sync_copy(x2T_hbm.at[b].at[pl.ds(x2base * NCP, 192 * NCP)], x2v)
        pltpu.sync_copy(selp_hbm.at[b], selv)
        pltpu.sync_copy(param_hbm, pv)

        def zrow(i, _):
            for u in range(8):
                yloc[pl.ds(i * 128 + u * 16, 16)] = zeros16
            return 0
        lax.fori_loop(0, SLAB_W // 128, zrow, 0)

        def mzero(i, _):
            for u in range(8):
                msmem[i * 8 + u] = 0.0
            return 0
        lax.fori_loop(0, ROWS // 8, mzero, 0)
        for j in range(NSELP // 16):
            idxv = selv[pl.ds(j * 16, 16)]
            for t in range(16):
                l = idxv[t] - start + HALO
                ok = (l >= 0) & (l < ROWS)
                msmem[jnp.where(ok, l, ROWS + 32)] = 1.0

        pb = HALO - podd * 128

        def csum(x, tot):
            return tot + jnp.where(msmem[pb + x] > 0.0, 1, 0)

        tpair = lax.fori_loop(0, 256, csum, 0)
        half = (tpair + 1) // 2

        def walk(x, st):
            pf, sstar = st
            pf = pf + jnp.where(msmem[pb + x] > 0.0, 1, 0)
            return pf, sstar + jnp.where(pf < half, 1, 0)

        _, sstar = lax.fori_loop(0, 160, walk, (0, 0))
        bnd = jnp.clip(sstar + 1, 100, 156)
        q_lo = jnp.where(podd == 0, 0, bnd - 128)
        q_hi = jnp.where(podd == 0, bnd, 128)

        def pix(q_local, _):
            q = start + q_local
            mq = msmem[q_local + HALO]

            @pl.when(mq > 0.0)
            def _():
                lrow = q_local + HALO
                fq = [ft[pl.ds(lrow * KP + j * 16, 16)] for j in range(NCH)]
                r = q // N
                c = jnp.remainder(q, N)

                def rowA(dri, cnt):
                    rr = r + dri - RAD
                    row_ok = (rr >= 0) & (rr < N)
                    l2row = q_local + HALO + (dri - RAD) * N - RAD
                    for dci in range(2 * RAD + 1):
                        cc = c + dci - RAD
                        l2 = l2row + dci
                        ok = row_ok & (cc >= 0) & (cc < N)
                        take = jnp.where(ok, msmem[l2], 0.0) > 0.0
                        lsm[jnp.where(take, 8 + cnt, 63)] = l2
                        cnt = cnt + jnp.where(take, 1, 0)
                    return cnt

                cnt = lax.fori_loop(0, 2 * RAD + 1, rowA, 0)
                lsm[0] = cnt

                def offA(i, rs):
                    l2 = lsm[8 + i]
                    fns = [ft[pl.ds(l2 * KP + j * 16, 16)]
                           for j in range(NCH)]
                    es = [jnp.exp(-jnp.abs(fq[j] - fns[j]))
                          for j in range(NCH)]
                    for j in range(NCH):
                        ev[pl.ds(i * KP + j * 16, 16)] = es[j]
                    return tuple(rs[j] + es[j] for j in range(NCH))

                rs = lax.fori_loop(0, cnt, offA,
                                   tuple(zeros16 for _ in range(NCH)))

                inv = [pv[pl.ds(j * 16, 16)] / rs[j] for j in range(NCH)]
                xrow = q_local + podd * 64
                x2a = x2v[pl.ds(xrow * NCP, 16)]
                x2b = x2v[pl.ds(xrow * NCP + 16, 16)]

                def offB(i, _c):
                    l2 = lsm[8 + i]
                    evs = [ev[pl.ds(i * KP + j * 16, 16)]
                           for j in range(NCH)]
                    prods = [evs[j] * inv[j] for j in range(NCH)]
                    while len(prods) > 1:
                        prods = ([a2 + b2 for a2, b2 in
                                  zip(prods[::2], prods[1::2])]
                                 + ([prods[-1]] if len(prods) % 2 else []))
                    acc = prods[0]
                    for p in perms:
                        acc = acc + lane_perm(acc, p)
                    plsc.addupdate(yloc.at[pl.ds(l2 * NCP, 16)], acc * x2a)
                    plsc.addupdate(yloc.at[pl.ds(l2 * NCP + 16, 16)], acc * x2b)
                    return _c

                lax.fori_loop(0, lsm[0], offB, 0)
            return 0

        lax.fori_loop(q_lo, q_hi, pix, 0)

        pltpu.sync_copy(yloc, ysh.at[pl.ds(sid * SLAB_W, SLAB_W)])
        plsc.subcore_barrier()

        lo = jnp.maximum(sid - 2, 0)
        hi = jnp.minimum(sid + 2, 15)
        pltpu.sync_copy(ysh.at[pl.ds(lo * SLAB_W + 2 * BLK_W, BLK_W)],
                        ybuf.at[pl.ds(0, BLK_W)])
        pltpu.sync_copy(ysh.at[pl.ds(hi * SLAB_W, BLK_W)],
                        ybuf.at[pl.ds(BLK_W, BLK_W)])
        has_lo = rb > 0
        has_hi = rb < NBLK - 1

        def addrow(i, _):
            for u in range(4):
                p = i * 64 + u * 16
                own = yloc[pl.ds(BLK_W + p, 16)]
                va = jnp.where(has_lo, ybuf[pl.ds(p, 16)], zeros16)
                vb = jnp.where(has_hi, ybuf[pl.ds(BLK_W + p, 16)], zeros16)
                yout[pl.ds(p, 16)] = own + va + vb
            return 0
        lax.fori_loop(0, BLK_W // 64, addrow, 0)

        pltpu.sync_copy(yout, out_hbm.at[b].at[pl.ds(start * NCP, BLK_W)])

    return k(featT, x2T, selp, param)


def kernel(x, conv1, conv2, seg_logits, conv_parameter, sel_idx):
    feat = jnp.concatenate([x, conv1, conv2], axis=1).reshape(B, K, N2)
    featT = jnp.transpose(feat, (0, 2, 1))
    featT = jnp.pad(featT, ((0, 0), (HALO, HALO), (0, KP - K)))
    featT = featT.reshape(B, (N2 + 2 * HALO) * KP)
    x2T = jnp.transpose(seg_logits.reshape(B, NC, N2), (0, 2, 1))
    x2T = jnp.pad(x2T, ((0, 0), (0, 0), (0, NCP - NC)))
    x2T = x2T.reshape(B, N2 * NCP)
    selp = jnp.concatenate(
        [sel_idx, jnp.tile(sel_idx[:, :1], (1, NSELP - NSEL))], axis=1)
    param = jnp.pad(conv_parameter, (0, KP - K))
    out = _sc_affinity(featT, x2T, selp.astype(jnp.int32), param)
    return out.reshape(B, N2, NCP)[:, :, :NC]
